# baseline (speedup 1.0000x reference)
import functools

import jax
import jax.numpy as jnp
from jax.experimental import pallas as pl
from jax.experimental.pallas import tpu as pltpu

K = 8
NUM_CLASSES = 100
INPUT_DIM = NUM_CLASSES + 1


def _loss_body(len_ref, pred_ref, lab_ref, ts_ref, out_ref):
    i = pl.program_id(0)

    @pl.when(i == 0)
    def _init():
        out_ref[...] = jnp.zeros_like(out_ref)

    Lm = 42
    x = pred_ref[:, :Lm, :]
    labs = lab_ref[...]
    ts = ts_ref[...]
    lens = len_ref[0, 0, :]

    BB = x.shape[0]
    pos = jax.lax.broadcasted_iota(jnp.int32, (BB, Lm), 1)
    maskf = (pos < lens[:, None]).astype(jnp.float32)

    ce_total = jnp.float32(0.0)
    mse_total = jnp.float32(0.0)
    for k in range(K):
        seg = x[:, :, k * INPUT_DIM : k * INPUT_DIM + NUM_CLASSES]
        tpred = x[:, :, k * INPUT_DIM + NUM_CLASSES]
        lw = labs[:, 1 + k : 1 + k + Lm]
        tw = ts[:, 1 + k : 1 + k + Lm]

        mx = jnp.max(seg, axis=-1)
        ex = jnp.exp(seg - mx[..., None])
        lse = jnp.log(jnp.sum(ex, axis=-1)) + mx
        lane = jax.lax.broadcasted_iota(jnp.int32, seg.shape, 2)
        tgt = jnp.sum(jnp.where(lane == lw[..., None], seg, 0.0), axis=-1)
        ce_total += jnp.sum((lse - tgt) * maskf)
        mse_total += jnp.sum((tpred - tw) ** 2 * maskf)

    lane = jax.lax.broadcasted_iota(jnp.int32, (1, 128), 1)
    upd = (
        jnp.where(lane == 0, ce_total, 0.0)
        + jnp.where(lane == 1, mse_total, 0.0)
        + jnp.where(lane == 2, jnp.sum(maskf), 0.0)
    )
    out_ref[...] += upd


@functools.partial(jax.jit, static_argnames=("bb",))
def _next_k_loss(predictions, labels, timestamps, seq_lens, bb=8):
    B, L, C = predictions.shape
    Lm = L - K
    lengths = jnp.clip(seq_lens - K, 0, Lm).astype(jnp.int32)
    lengths3 = lengths.reshape(B // bb, 1, bb)

    grid = (B // bb,)
    out = pl.pallas_call(
        _loss_body,
        grid=grid,
        in_specs=[
            pl.BlockSpec((1, 1, bb), lambda i: (i, 0, 0)),
            pl.BlockSpec((bb, 48, C), lambda i: (i, 0, 0)),
            pl.BlockSpec((bb, L), lambda i: (i, 0)),
            pl.BlockSpec((bb, L), lambda i: (i, 0)),
        ],
        out_specs=pl.BlockSpec((1, 128), lambda i: (0, 0)),
        out_shape=jax.ShapeDtypeStruct((1, 128), jnp.float32),
    )(lengths3, predictions, labels.astype(jnp.int32), timestamps)

    ce_sum = out[0, 0]
    mse_sum = out[0, 1]
    denom = jnp.maximum(out[0, 2] * K, 1.0)
    return jnp.stack([ce_sum / denom, mse_sum / denom])


def kernel(predictions, labels, timestamps, seq_lens):
    return _next_k_loss(predictions, labels, timestamps, seq_lens)


# speedup vs baseline: 1.3440x; 1.3440x over previous
import functools

import jax
import jax.numpy as jnp
from jax.experimental import pallas as pl
from jax.experimental.pallas import tpu as pltpu

K = 8
NUM_CLASSES = 100
INPUT_DIM = NUM_CLASSES + 1
LM = 42


def _loss_body(len_ref, pred_ref, lab_ref, ts_ref, out_ref):
    i = pl.program_id(0)

    @pl.when(i == 0)
    def _init():
        out_ref[...] = jnp.zeros_like(out_ref)

    x = pred_ref[:, :LM]
    labs = lab_ref[...]
    ts = ts_ref[...]
    lens = len_ref[0, 0, :]

    BB = x.shape[0]
    pos = jax.lax.broadcasted_iota(jnp.int32, (BB, LM), 1)
    maskf = (pos < lens[:, None]).astype(jnp.float32)

    lane = jax.lax.broadcasted_iota(jnp.int32, x.shape, 3)
    classmask = lane < NUM_CLASSES
    neg_inf = jnp.float32(-jnp.inf)

    mx = jnp.max(jnp.where(classmask, x, neg_inf), axis=-1)
    ex = jnp.where(classmask, jnp.exp(x - mx[..., None]), 0.0)
    lse = jnp.log(jnp.sum(ex, axis=-1)) + mx

    lw = jnp.stack([labs[:, 1 + k : 1 + k + LM] for k in range(K)], axis=-1)
    tw = jnp.stack([ts[:, 1 + k : 1 + k + LM] for k in range(K)], axis=-1)

    tgt = jnp.sum(jnp.where(lane == lw[..., None], x, 0.0), axis=-1)
    tpred = x[..., NUM_CLASSES]

    w = maskf[..., None]
    ce_total = jnp.sum((lse - tgt) * w)
    mse_total = jnp.sum((tpred - tw) ** 2 * w)

    olane = jax.lax.broadcasted_iota(jnp.int32, (1, 128), 1)
    upd = (
        jnp.where(olane == 0, ce_total, 0.0)
        + jnp.where(olane == 1, mse_total, 0.0)
        + jnp.where(olane == 2, jnp.sum(maskf), 0.0)
    )
    out_ref[...] += upd


@functools.partial(jax.jit, static_argnames=("bb",))
def _next_k_loss(predictions, labels, timestamps, seq_lens, bb=8):
    B, L, C = predictions.shape
    preds4 = predictions.reshape(B, L, K, INPUT_DIM)
    lengths = jnp.clip(seq_lens - K, 0, LM).astype(jnp.int32)
    lengths3 = lengths.reshape(B // bb, 1, bb)

    grid = (B // bb,)
    out = pl.pallas_call(
        _loss_body,
        grid=grid,
        in_specs=[
            pl.BlockSpec((1, 1, bb), lambda i: (i, 0, 0)),
            pl.BlockSpec((bb, 48, K, INPUT_DIM), lambda i: (i, 0, 0, 0)),
            pl.BlockSpec((bb, L), lambda i: (i, 0)),
            pl.BlockSpec((bb, L), lambda i: (i, 0)),
        ],
        out_specs=pl.BlockSpec((1, 128), lambda i: (0, 0)),
        out_shape=jax.ShapeDtypeStruct((1, 128), jnp.float32),
    )(lengths3, preds4, labels.astype(jnp.int32), timestamps)

    ce_sum = out[0, 0]
    mse_sum = out[0, 1]
    denom = jnp.maximum(out[0, 2] * K, 1.0)
    return jnp.stack([ce_sum / denom, mse_sum / denom])


def kernel(predictions, labels, timestamps, seq_lens):
    return _next_k_loss(predictions, labels, timestamps, seq_lens)


# speedup vs baseline: 1.8627x; 1.3859x over previous
import functools

import jax
import jax.numpy as jnp
from jax import lax
from jax.experimental import pallas as pl
from jax.experimental.pallas import tpu as pltpu

K = 8
NUM_CLASSES = 100
INPUT_DIM = NUM_CLASSES + 1
LM = 42


def _loss_body(len_ref, pred_ref, lab_ref, ts_ref, out_ref):
    i = pl.program_id(0)

    @pl.when(i == 0)
    def _init():
        out_ref[...] = jnp.zeros_like(out_ref)

    x4 = pred_ref[:, :LM]
    labs = lab_ref[...]
    ts = ts_ref[...]
    lens = len_ref[0, 0, :]

    BB = x4.shape[0]
    N = BB * LM * K
    pos = jax.lax.broadcasted_iota(jnp.int32, (BB, LM), 1)
    maskf = (pos < lens[:, None]).astype(jnp.float32)
    wk = jnp.broadcast_to(maskf[:, :, None], (BB, LM, K))

    lw = jnp.stack([labs[:, 1 + k : 1 + k + LM] for k in range(K)], axis=-1)
    tw = jnp.stack([ts[:, 1 + k : 1 + k + LM] for k in range(K)], axis=-1)
    lw = jnp.where(wk > 0, lw, -1)

    lane4 = jax.lax.broadcasted_iota(jnp.int32, x4.shape, 3)

    ones_cls = jnp.where(
        jax.lax.broadcasted_iota(jnp.int32, (INPUT_DIM, 128), 0) < NUM_CLASSES,
        jnp.float32(1.0),
        jnp.float32(0.0),
    )
    ex = jnp.exp(x4).reshape(N, INPUT_DIM)
    s = lax.dot_general(
        ex, ones_cls, (((1,), (0,)), ((), ())), preferred_element_type=jnp.float32
    )
    s4 = s.reshape(BB, LM, K, 128)
    lse_sum = jnp.sum(jnp.log(s4) * wk[..., None]) * jnp.float32(1.0 / 128.0)

    tgt_sum = jnp.sum(jnp.where(lane4 == lw[..., None], x4, 0.0))

    sq = (x4 - tw[..., None]) ** 2
    mse_sum = jnp.sum(
        jnp.where(lane4 == NUM_CLASSES, sq, 0.0) * wk[..., None]
    )

    cnt = jnp.sum(maskf)
    ce_total = lse_sum - tgt_sum

    olane = jax.lax.broadcasted_iota(jnp.int32, (1, 128), 1)
    upd = (
        jnp.where(olane == 0, ce_total, 0.0)
        + jnp.where(olane == 1, mse_sum, 0.0)
        + jnp.where(olane == 2, cnt, 0.0)
    )
    out_ref[...] += upd


@functools.partial(jax.jit, static_argnames=("bb",))
def _next_k_loss(predictions, labels, timestamps, seq_lens, bb=8):
    B, L, C = predictions.shape
    preds4 = predictions.reshape(B, L, K, INPUT_DIM)
    lengths = jnp.clip(seq_lens - K, 0, LM).astype(jnp.int32)
    lengths3 = lengths.reshape(B // bb, 1, bb)

    grid = (B // bb,)
    out = pl.pallas_call(
        _loss_body,
        grid=grid,
        in_specs=[
            pl.BlockSpec((1, 1, bb), lambda i: (i, 0, 0)),
            pl.BlockSpec((bb, 48, K, INPUT_DIM), lambda i: (i, 0, 0, 0)),
            pl.BlockSpec((bb, L), lambda i: (i, 0)),
            pl.BlockSpec((bb, L), lambda i: (i, 0)),
        ],
        out_specs=pl.BlockSpec((1, 128), lambda i: (0, 0)),
        out_shape=jax.ShapeDtypeStruct((1, 128), jnp.float32),
    )(lengths3, preds4, labels.astype(jnp.int32), timestamps)

    ce_sum = out[0, 0]
    mse_sum = out[0, 1]
    denom = jnp.maximum(out[0, 2] * K, 1.0)
    return jnp.stack([ce_sum / denom, mse_sum / denom])


def kernel(predictions, labels, timestamps, seq_lens):
    return _next_k_loss(predictions, labels, timestamps, seq_lens)


# speedup vs baseline: 2.5047x; 1.3447x over previous
import functools

import jax
import jax.numpy as jnp
from jax import lax
from jax.experimental import pallas as pl
from jax.experimental.pallas import tpu as pltpu

K = 8
NUM_CLASSES = 100
INPUT_DIM = NUM_CLASSES + 1
C = K * INPUT_DIM
LM = 42
LP = 48


def _loss_body(len_ref, pred_ref, lw_ref, tw_ref, out_ref):
    i = pl.program_id(0)

    @pl.when(i == 0)
    def _init():
        out_ref[...] = jnp.zeros_like(out_ref)

    BB = pred_ref.shape[0]
    NR = BB * LP

    x = pred_ref[...].reshape(NR, C)
    lw8 = lw_ref[...].reshape(NR, K)
    tw8 = tw_ref[...].reshape(NR, K)
    lenr = len_ref[...].reshape(NR, 1)

    rowi = jax.lax.broadcasted_iota(jnp.int32, (NR, 1), 0)
    posr = rowi - (rowi // LP) * LP
    validf = (posr < lenr).astype(jnp.float32)

    jlane = jax.lax.broadcasted_iota(jnp.int32, (K, C), 1)
    trow = jax.lax.broadcasted_iota(jnp.int32, (K, C), 0)
    jseg = jlane // INPUT_DIM
    jcls = jlane - jseg * INPUT_DIM
    e2 = (jseg == trow).astype(jnp.float32)
    e1 = e2 * (jcls < NUM_CLASSES).astype(jnp.float32)

    ex = jnp.exp(x)
    s8 = lax.dot_general(
        ex, e1, (((1,), (1,)), ((), ())), preferred_element_type=jnp.float32
    )
    lse_sum = jnp.sum(jnp.log(s8) * validf)

    ki = jax.lax.broadcasted_iota(jnp.int32, (NR, K), 1)
    tlane = jnp.where(validf > 0, ki * INPUT_DIM + lw8, -1).astype(jnp.float32)
    t_exp = lax.dot_general(
        tlane, e2, (((1,), (0,)), ((), ())), preferred_element_type=jnp.float32
    )
    lanei = jax.lax.broadcasted_iota(jnp.int32, (NR, C), 1)
    tgt_sum = jnp.sum(jnp.where(lanei == t_exp.astype(jnp.int32), x, 0.0))

    tw_exp = lax.dot_general(
        tw8, e2, (((1,), (0,)), ((), ())), preferred_element_type=jnp.float32
    )
    iseg = lanei // INPUT_DIM
    is_time = (lanei - iseg * INPUT_DIM) == NUM_CLASSES
    d = x - tw_exp
    mse_sum = jnp.sum(jnp.where(is_time, d * d, 0.0) * validf)

    cnt = jnp.sum(validf)
    ce_total = lse_sum - tgt_sum

    olane = jax.lax.broadcasted_iota(jnp.int32, (1, 128), 1)
    upd = (
        jnp.where(olane == 0, ce_total, 0.0)
        + jnp.where(olane == 1, mse_sum, 0.0)
        + jnp.where(olane == 2, cnt, 0.0)
    )
    out_ref[...] += upd


@functools.partial(jax.jit, static_argnames=("bb",))
def _next_k_loss(predictions, labels, timestamps, seq_lens, bb=8):
    B, L, _ = predictions.shape
    lengths = jnp.clip(seq_lens - K, 0, LM).astype(jnp.int32)

    labp = jnp.concatenate(
        [labels.astype(jnp.int32), jnp.zeros((B, LP + K - L), jnp.int32)], axis=1
    )
    tsp = jnp.concatenate(
        [timestamps, jnp.zeros((B, LP + K - L), jnp.float32)], axis=1
    )
    widx = jnp.arange(LP)[:, None] + 1 + jnp.arange(K)[None, :]
    lw = labp[:, widx]
    tw = tsp[:, widx]
    lenexp = jnp.broadcast_to(lengths[:, None, None], (B, LP, 1))

    grid = (B // bb,)
    out = pl.pallas_call(
        _loss_body,
        grid=grid,
        in_specs=[
            pl.BlockSpec((bb, LP, 1), lambda i: (i, 0, 0)),
            pl.BlockSpec((bb, LP, C), lambda i: (i, 0, 0)),
            pl.BlockSpec((bb, LP, K), lambda i: (i, 0, 0)),
            pl.BlockSpec((bb, LP, K), lambda i: (i, 0, 0)),
        ],
        out_specs=pl.BlockSpec((1, 128), lambda i: (0, 0)),
        out_shape=jax.ShapeDtypeStruct((1, 128), jnp.float32),
    )(lenexp, predictions, lw, tw)

    ce_sum = out[0, 0]
    mse_sum = out[0, 1]
    denom = jnp.maximum(out[0, 2] * K, 1.0)
    return jnp.stack([ce_sum / denom, mse_sum / denom])


def kernel(predictions, labels, timestamps, seq_lens):
    return _next_k_loss(predictions, labels, timestamps, seq_lens)


# speedup vs baseline: 2.8146x; 1.1237x over previous
import functools

import jax
import jax.numpy as jnp
from jax import lax
from jax.experimental import pallas as pl
from jax.experimental.pallas import tpu as pltpu

K = 8
NUM_CLASSES = 100
INPUT_DIM = NUM_CLASSES + 1
C = K * INPUT_DIM
LM = 42
LP = 48


def _loss_body(len_ref, pred_ref, lw_ref, tw_ref, out_ref):
    i = pl.program_id(0)

    @pl.when(i == 0)
    def _init():
        out_ref[...] = jnp.zeros_like(out_ref)

    BB = pred_ref.shape[0]
    NR = BB * LP

    x = pred_ref[...].reshape(NR, C)
    lw8 = lw_ref[...].reshape(NR, K)
    tw8 = tw_ref[...].reshape(NR, K)
    lenr = len_ref[...].reshape(NR, 1)

    rowi = jax.lax.broadcasted_iota(jnp.int32, (NR, 1), 0)
    posr = rowi - (rowi // LP) * LP
    validf = (posr < lenr).astype(jnp.float32)

    jlane = jax.lax.broadcasted_iota(jnp.int32, (K, C), 1)
    trow = jax.lax.broadcasted_iota(jnp.int32, (K, C), 0)
    jseg = jlane // INPUT_DIM
    jcls = jlane - jseg * INPUT_DIM
    e2 = (jseg == trow).astype(jnp.float32)
    e1 = e2 * (jcls < NUM_CLASSES).astype(jnp.float32)

    ex = jnp.exp(x)
    s8 = lax.dot_general(
        ex, e1, (((1,), (1,)), ((), ())), preferred_element_type=jnp.float32
    )
    lse_sum = jnp.sum(jnp.log(s8) * validf)

    ki = jax.lax.broadcasted_iota(jnp.int32, (NR, K), 1)
    tlane = jnp.where(validf > 0, ki * INPUT_DIM + lw8, -1).astype(jnp.float32)
    t_exp = lax.dot_general(
        tlane, e2, (((1,), (0,)), ((), ())), preferred_element_type=jnp.float32
    )
    lanei = jax.lax.broadcasted_iota(jnp.int32, (NR, C), 1)
    tgt_sum = jnp.sum(jnp.where(lanei == t_exp.astype(jnp.int32), x, 0.0))

    tw_exp = lax.dot_general(
        tw8, e2, (((1,), (0,)), ((), ())), preferred_element_type=jnp.float32
    )
    iseg = lanei // INPUT_DIM
    is_time = (lanei - iseg * INPUT_DIM) == NUM_CLASSES
    d = x - tw_exp
    mse_sum = jnp.sum(jnp.where(is_time, d * d, 0.0) * validf)

    cnt = jnp.sum(validf)
    ce_total = lse_sum - tgt_sum

    olane = jax.lax.broadcasted_iota(jnp.int32, (1, 128), 1)
    upd = (
        jnp.where(olane == 0, ce_total, 0.0)
        + jnp.where(olane == 1, mse_sum, 0.0)
        + jnp.where(olane == 2, cnt, 0.0)
    )
    out_ref[...] += upd


@functools.partial(jax.jit, static_argnames=("bb",))
def _next_k_loss(predictions, labels, timestamps, seq_lens, bb=32):
    B, L, _ = predictions.shape
    lengths = jnp.clip(seq_lens - K, 0, LM).astype(jnp.int32)

    labp = jnp.concatenate(
        [labels.astype(jnp.int32), jnp.zeros((B, LP + K - L), jnp.int32)], axis=1
    )
    tsp = jnp.concatenate(
        [timestamps, jnp.zeros((B, LP + K - L), jnp.float32)], axis=1
    )
    widx = jnp.arange(LP)[:, None] + 1 + jnp.arange(K)[None, :]
    lw = labp[:, widx]
    tw = tsp[:, widx]
    lenexp = jnp.broadcast_to(lengths[:, None, None], (B, LP, 1))

    grid = (B // bb,)
    out = pl.pallas_call(
        _loss_body,
        grid=grid,
        in_specs=[
            pl.BlockSpec((bb, LP, 1), lambda i: (i, 0, 0)),
            pl.BlockSpec((bb, LP, C), lambda i: (i, 0, 0)),
            pl.BlockSpec((bb, LP, K), lambda i: (i, 0, 0)),
            pl.BlockSpec((bb, LP, K), lambda i: (i, 0, 0)),
        ],
        out_specs=pl.BlockSpec((1, 128), lambda i: (0, 0)),
        out_shape=jax.ShapeDtypeStruct((1, 128), jnp.float32),
    )(lenexp, predictions, lw, tw)

    ce_sum = out[0, 0]
    mse_sum = out[0, 1]
    denom = jnp.maximum(out[0, 2] * K, 1.0)
    return jnp.stack([ce_sum / denom, mse_sum / denom])


def kernel(predictions, labels, timestamps, seq_lens):
    return _next_k_loss(predictions, labels, timestamps, seq_lens)
